# baseline (speedup 1.0000x reference)
import functools

import jax
import jax.numpy as jnp
from jax import lax
from jax.experimental import pallas as pl
from jax.experimental.pallas import tpu as pltpu
from jax.experimental.pallas import tpu_sc as plsc

NC = 2
NS = 16
NW = NC * NS
CHUNK = 128
PASSES = 4
BLK = 2000


def _zero_fill(ref, nrows, width):
  zeros16 = jnp.zeros((16,), jnp.float32)

  def body(i, _):
    for j in range(width // 16):
      ref[i, pl.ds(j * 16, 16)] = zeros16
    return 0

  lax.fori_loop(0, nrows, body, 0)


def _fill_ones(ref, nrows, width):
  ones16 = jnp.ones((16,), jnp.float32)

  def body(i, _):
    for j in range(width // 16):
      ref[i, pl.ds(j * 16, 16)] = ones16
    return 0

  lax.fori_loop(0, nrows, body, 0)


@functools.lru_cache(maxsize=None)
def _make_deg_kernel(cht, ch0, z_rows):
  rows_per_tile = z_rows // NS
  mesh = plsc.VectorSubcoreMesh(core_axis_name="c", subcore_axis_name="s")

  @functools.partial(
      pl.kernel,
      out_type=jax.ShapeDtypeStruct((NC, z_rows, 16), jnp.float32),
      mesh=mesh,
      scratch_types=[
          pltpu.VMEM((cht, CHUNK), jnp.int32),
          pltpu.VMEM((CHUNK, 16), jnp.float32),
          pltpu.VMEM_SHARED((z_rows, 16), jnp.float32),
          pltpu.SemaphoreType.DMA,
      ],
      compiler_params=pltpu.CompilerParams(use_tc_tiling_on_sc=False),
  )
  def deg_kernel(coli_hbm, out_hbm, coli_v, ones_v, hsh, sem):
    cid = lax.axis_index("c")
    sid = lax.axis_index("s")
    base = jnp.where(cid == 0, 0, ch0)
    cnt = jnp.where(cid == 0, ch0, cht - ch0)
    _zero_fill(ones_v, CHUNK, 16)
    for k in range(rows_per_tile // CHUNK):
      pltpu.sync_copy(ones_v, hsh.at[pl.ds(sid * rows_per_tile + k * CHUNK, CHUNK)])
    _fill_ones(ones_v, CHUNK, 16)
    pltpu.sync_copy(coli_hbm.at[sid], coli_v)
    plsc.subcore_barrier()

    def body(jj, _):
      pltpu.sync_copy(ones_v, hsh.at[coli_v.at[base + jj]], add=True)
      return 0

    lax.fori_loop(0, cnt, body, 0)
    plsc.subcore_barrier()
    pltpu.sync_copy(
        hsh.at[pl.ds(sid * rows_per_tile, rows_per_tile)],
        out_hbm.at[cid, pl.ds(sid * rows_per_tile, rows_per_tile)],
    )

  return deg_kernel


@functools.lru_cache(maxsize=None)
def _make_agg_kernel(n_rows, d, cht, ch0, z_rows):
  pw = d // PASSES
  rows_per_tile = z_rows // NS
  y_rows_per_tile = n_rows // NS
  mesh = plsc.VectorSubcoreMesh(core_axis_name="c", subcore_axis_name="s")

  @functools.partial(
      pl.kernel,
      out_type=jax.ShapeDtypeStruct((NC, z_rows, d), jnp.float32),
      mesh=mesh,
      scratch_types=[
          pltpu.VMEM((cht, CHUNK), jnp.int32),
          pltpu.VMEM((cht, CHUNK), jnp.int32),
          pltpu.VMEM((CHUNK, pw), jnp.float32),
          pltpu.VMEM((CHUNK, pw), jnp.float32),
          pltpu.VMEM((CHUNK, pw), jnp.float32),
          pltpu.VMEM((CHUNK, pw), jnp.float32),
          pltpu.VMEM((CHUNK, pw), jnp.float32),
          pltpu.VMEM_SHARED((z_rows, pw), jnp.float32),
          pltpu.VMEM_SHARED((n_rows, pw), jnp.float32),
          pltpu.SemaphoreType.DMA,
          pltpu.SemaphoreType.DMA,
          [pltpu.SemaphoreType.DMA] * 4,
          [pltpu.SemaphoreType.DMA] * 4,
      ],
      compiler_params=pltpu.CompilerParams(use_tc_tiling_on_sc=False),
  )
  def agg_kernel(y_hbm, rowi_hbm, coli_hbm, out_hbm,
                 rowi_v, coli_v, zbuf, gb0, gb1, gb2, gb3, zsh, ysh,
                 ysem, wsem, gsems, ssems):
    gbufs = (gb0, gb1, gb2, gb3)
    cid = lax.axis_index("c")
    sid = lax.axis_index("s")
    pltpu.async_copy(rowi_hbm.at[sid], rowi_v, gsems[0])
    pltpu.async_copy(coli_hbm.at[sid], coli_v, gsems[1])
    _zero_fill(zbuf, CHUNK, pw)
    pltpu.make_async_copy(rowi_hbm.at[sid], rowi_v, gsems[0]).wait()
    pltpu.make_async_copy(coli_hbm.at[sid], coli_v, gsems[1]).wait()
    base = jnp.where(cid == 0, 0, ch0)
    cnt = jnp.where(cid == 0, ch0, cht - ch0)

    k_hbm = jnp.where(cid == 0, round(0.4 * ch0), round(0.22 * (cht - ch0)))

    def gather_wait(j, b):
      pltpu.make_async_copy(ysh.at[rowi_v.at[j]], gbufs[b], gsems[b]).wait()

    def scatter(j, b):
      pltpu.async_copy(gbufs[b], zsh.at[coli_v.at[j]], ssems[b], add=True)

    def scatter_wait(b):
      pltpu.make_async_copy(gbufs[b], zsh.at[coli_v.at[base]], ssems[b]).wait()

    zrow0 = sid * rows_per_tile
    out_rows = out_hbm.at[cid, pl.ds(zrow0, rows_per_tile)]
    zsh_rows = zsh.at[pl.ds(zrow0, rows_per_tile)]

    for p in range(PASSES):
      y_src = y_hbm.at[p, pl.ds(sid * y_rows_per_tile, y_rows_per_tile)]
      y_dst = ysh.at[pl.ds(sid * y_rows_per_tile, y_rows_per_tile)]
      pltpu.async_copy(y_src, y_dst, ysem)
      if p > 0:
        pltpu.make_async_copy(
            zsh_rows, out_rows.at[:, pl.ds((p - 1) * pw, pw)], wsem).wait()
      for k in range(rows_per_tile // CHUNK):
        pltpu.sync_copy(zbuf, zsh.at[pl.ds(zrow0 + k * CHUNK, CHUNK)])
      pltpu.make_async_copy(y_src, y_dst, ysem).wait()
      plsc.subcore_barrier()

      def gather(jj, j, b, p=p):
        @pl.when(jj < k_hbm)
        def _():
          pltpu.async_copy(y_hbm.at[p].at[rowi_v.at[j]], gbufs[b], gsems[b])

        @pl.when(jj >= k_hbm)
        def _():
          pltpu.async_copy(ysh.at[rowi_v.at[j]], gbufs[b], gsems[b])

      gather(0, base, 0)

      @pl.when(cnt > 1)
      def _():
        gather(1, base + 1, 1)

      def body(jj, _):
        j = base + jj

        def step(b):
          bn = (b + 2) % 4
          gather_wait(j, b)
          scatter(j, b)

          @pl.when(jj + 2 < cnt)
          def _():
            @pl.when(jj >= 2)
            def _():
              scatter_wait(bn)

            gather(jj + 2, j + 2, bn)

        for b in range(4):
          @pl.when(jj % 4 == b)
          def _(b=b):
            step(b)

        return 0

      lax.fori_loop(0, cnt, body, 0)
      for b in range(4):
        @pl.when(cnt > b)
        def _(b=b):
          scatter_wait(b)

      plsc.subcore_barrier()
      pltpu.async_copy(zsh_rows, out_rows.at[:, pl.ds(p * pw, pw)], wsem)

    pltpu.make_async_copy(
        zsh_rows, out_rows.at[:, pl.ds((PASSES - 1) * pw, pw)], wsem).wait()

  return agg_kernel


def _split_panels(y_ref, y):
  pw = y.shape[1] // PASSES
  for k in range(PASSES):
    y_ref[k] = y[:, k * pw:(k + 1) * pw]


def _cat_panels(y_ref):
  return jnp.concatenate([y_ref[k] for k in range(PASSES)], axis=1)


def _stage_b_body(hist_ref, x_ref, w_ref, y_ref, dinv_ref):
  deg = hist_ref[0] + hist_ref[1] + 1.0
  dinv = lax.rsqrt(deg)
  y = jnp.dot(x_ref[...], w_ref[...], preferred_element_type=jnp.float32)
  _split_panels(y_ref, y * dinv[:, 0:1])
  dinv_ref[...] = dinv


def _stage_d_body(z_ref, y1_ref, dinv_ref, w_ref, b_ref, y2_ref):
  d = dinv_ref[...][:, 0:1]
  agg = z_ref[0] + z_ref[1] + _cat_panels(y1_ref)
  h = jnp.maximum(agg * d + b_ref[...], 0.0)
  y2 = jnp.dot(h, w_ref[...], preferred_element_type=jnp.float32) * d
  _split_panels(y2_ref, y2)


def _stage_f_body(z_ref, y2_ref, dinv_ref, b_ref, o_ref):
  d = dinv_ref[...][:, 0:1]
  o_ref[...] = (z_ref[0] + z_ref[1] + _cat_panels(y2_ref)) * d + b_ref[...]


def kernel(x, edge_index, W1, b1, W2, b2):
  n, d_in = x.shape
  d_hid = W1.shape[1]
  d_out = W2.shape[1]
  e = edge_index.shape[1]

  cht = -(-e // (NS * CHUNK))
  ch0 = round(cht / 2)
  e_pad = NS * cht * CHUNK
  z_rows = -(-(n + 1) // (NS * CHUNK)) * NS * CHUNK

  row = edge_index[0].astype(jnp.int32)
  col = edge_index[1].astype(jnp.int32)
  pad = e_pad - e
  row_p = jnp.concatenate([row, jnp.zeros((pad,), jnp.int32)]).reshape(NS, cht, CHUNK)
  col_p = jnp.concatenate([col, jnp.full((pad,), n, jnp.int32)]).reshape(NS, cht, CHUNK)

  hist = _make_deg_kernel(cht, ch0, z_rows)(col_p)

  nblk = -(-n // BLK)
  pw = d_hid // PASSES
  full_spec = pl.BlockSpec((BLK, d_hid), lambda i: (i, 0))
  panel_spec = pl.BlockSpec((PASSES, BLK, pw), lambda i: (0, i, 0))
  panel_shape = jax.ShapeDtypeStruct((PASSES, n, pw), jnp.float32)
  z_spec = pl.BlockSpec((NC, BLK, d_hid), lambda i: (0, i, 0))
  dinv_spec = pl.BlockSpec((BLK, 16), lambda i: (i, 0))

  y1, dinv = pl.pallas_call(
      _stage_b_body,
      grid=(nblk,),
      in_specs=[
          pl.BlockSpec((NC, BLK, 16), lambda i: (0, i, 0)),
          pl.BlockSpec((BLK, d_in), lambda i: (i, 0)),
          pl.BlockSpec((d_in, d_hid), lambda i: (0, 0)),
      ],
      out_specs=[panel_spec, dinv_spec],
      out_shape=[
          panel_shape,
          jax.ShapeDtypeStruct((n, 16), jnp.float32),
      ],
  )(hist, x, W1)

  agg_fn = _make_agg_kernel(n, d_hid, cht, ch0, z_rows)
  z1 = agg_fn(y1, row_p, col_p)

  y2 = pl.pallas_call(
      _stage_d_body,
      grid=(nblk,),
      in_specs=[
          z_spec, panel_spec, dinv_spec,
          pl.BlockSpec((d_hid, d_out), lambda i: (0, 0)),
          pl.BlockSpec((1, d_out), lambda i: (0, 0)),
      ],
      out_specs=panel_spec,
      out_shape=panel_shape,
  )(z1, y1, dinv, W2, b1.reshape(1, -1))

  z2 = agg_fn(y2, row_p, col_p)

  out = pl.pallas_call(
      _stage_f_body,
      grid=(nblk,),
      in_specs=[
          z_spec, panel_spec, dinv_spec,
          pl.BlockSpec((1, d_out), lambda i: (0, 0)),
      ],
      out_specs=full_spec,
      out_shape=jax.ShapeDtypeStruct((n, d_out), jnp.float32),
  )(z2, y2, dinv, b2.reshape(1, -1))

  return out


# speedup vs baseline: 1.2308x; 1.2308x over previous
import functools

import jax
import jax.numpy as jnp
from jax import lax
from jax.experimental import pallas as pl
from jax.experimental.pallas import tpu as pltpu
from jax.experimental.pallas import tpu_sc as plsc

NC = 2
NS = 16
NW = NC * NS
CHUNK = 128
PASSES = 4
BLK = 2000


def _zero_fill(ref, nrows, width):
  zeros16 = jnp.zeros((16,), jnp.float32)

  def body(i, _):
    for j in range(width // 16):
      ref[i, pl.ds(j * 16, 16)] = zeros16
    return 0

  lax.fori_loop(0, nrows, body, 0)


def _fill_ones(ref, nrows, width):
  ones16 = jnp.ones((16,), jnp.float32)

  def body(i, _):
    for j in range(width // 16):
      ref[i, pl.ds(j * 16, 16)] = ones16
    return 0

  lax.fori_loop(0, nrows, body, 0)


@functools.lru_cache(maxsize=None)
def _make_deg_kernel(cht, ch0, z_rows):
  rows_per_tile = z_rows // NS
  mesh = plsc.VectorSubcoreMesh(core_axis_name="c", subcore_axis_name="s")

  @functools.partial(
      pl.kernel,
      out_type=jax.ShapeDtypeStruct((NC, z_rows, 16), jnp.float32),
      mesh=mesh,
      scratch_types=[
          pltpu.VMEM((cht, CHUNK), jnp.int32),
          pltpu.VMEM((CHUNK, 16), jnp.float32),
          pltpu.VMEM_SHARED((z_rows, 16), jnp.float32),
          pltpu.SemaphoreType.DMA,
      ],
      compiler_params=pltpu.CompilerParams(use_tc_tiling_on_sc=False),
  )
  def deg_kernel(coli_hbm, out_hbm, coli_v, ones_v, hsh, sem):
    cid = lax.axis_index("c")
    sid = lax.axis_index("s")
    base = jnp.where(cid == 0, 0, ch0)
    cnt = jnp.where(cid == 0, ch0, cht - ch0)
    _zero_fill(ones_v, CHUNK, 16)
    for k in range(rows_per_tile // CHUNK):
      pltpu.sync_copy(ones_v, hsh.at[pl.ds(sid * rows_per_tile + k * CHUNK, CHUNK)])
    _fill_ones(ones_v, CHUNK, 16)
    pltpu.sync_copy(coli_hbm.at[sid], coli_v)
    plsc.subcore_barrier()

    def body(jj, _):
      pltpu.sync_copy(ones_v, hsh.at[coli_v.at[base + jj]], add=True)
      return 0

    lax.fori_loop(0, cnt, body, 0)
    plsc.subcore_barrier()
    pltpu.sync_copy(
        hsh.at[pl.ds(sid * rows_per_tile, rows_per_tile)],
        out_hbm.at[cid, pl.ds(sid * rows_per_tile, rows_per_tile)],
    )

  return deg_kernel


@functools.lru_cache(maxsize=None)
def _make_agg_kernel(n_rows, d, cht, ch0, z_rows):
  pw = d // PASSES
  rows_per_tile = z_rows // NS
  y_rows_per_tile = n_rows // NS
  mesh = plsc.VectorSubcoreMesh(core_axis_name="c", subcore_axis_name="s")

  @functools.partial(
      pl.kernel,
      out_type=jax.ShapeDtypeStruct((NC, z_rows, d), jnp.float32),
      mesh=mesh,
      scratch_types=[
          pltpu.VMEM((cht, CHUNK), jnp.int32),
          pltpu.VMEM((cht, CHUNK), jnp.int32),
          pltpu.VMEM((CHUNK, pw), jnp.float32),
          pltpu.VMEM((CHUNK, pw), jnp.float32),
          pltpu.VMEM((CHUNK, pw), jnp.float32),
          pltpu.VMEM((CHUNK, pw), jnp.float32),
          pltpu.VMEM((CHUNK, pw), jnp.float32),
          pltpu.VMEM_SHARED((z_rows, pw), jnp.float32),
          pltpu.VMEM_SHARED((n_rows, pw), jnp.float32),
          pltpu.SemaphoreType.DMA,
          pltpu.SemaphoreType.DMA,
          [pltpu.SemaphoreType.DMA] * 4,
          [pltpu.SemaphoreType.DMA] * 4,
      ],
      compiler_params=pltpu.CompilerParams(use_tc_tiling_on_sc=False),
  )
  def agg_kernel(y_hbm, rowi_hbm, coli_hbm, out_hbm,
                 rowi_v, coli_v, zbuf, gb0, gb1, gb2, gb3, zsh, ysh,
                 ysem, wsem, gsems, ssems):
    gbufs = (gb0, gb1, gb2, gb3)
    cid = lax.axis_index("c")
    sid = lax.axis_index("s")
    pltpu.async_copy(rowi_hbm.at[sid], rowi_v, gsems[0])
    pltpu.async_copy(coli_hbm.at[sid], coli_v, gsems[1])
    _zero_fill(zbuf, CHUNK, pw)
    pltpu.make_async_copy(rowi_hbm.at[sid], rowi_v, gsems[0]).wait()
    pltpu.make_async_copy(coli_hbm.at[sid], coli_v, gsems[1]).wait()
    base = jnp.where(cid == 0, 0, ch0)
    cnt = jnp.where(cid == 0, ch0, cht - ch0)

    def gather(j, b):
      pltpu.async_copy(ysh.at[rowi_v.at[j]], gbufs[b], gsems[b])

    def gather_wait(j, b):
      pltpu.make_async_copy(ysh.at[rowi_v.at[j]], gbufs[b], gsems[b]).wait()

    def scatter(j, b):
      pltpu.async_copy(gbufs[b], zsh.at[coli_v.at[j]], ssems[b], add=True)

    def scatter_wait(b):
      pltpu.make_async_copy(gbufs[b], zsh.at[coli_v.at[base]], ssems[b]).wait()

    zrow0 = sid * rows_per_tile
    out_rows = out_hbm.at[cid, pl.ds(zrow0, rows_per_tile)]
    zsh_rows = zsh.at[pl.ds(zrow0, rows_per_tile)]

    for p in range(PASSES):
      y_src = y_hbm.at[pl.ds(sid * y_rows_per_tile, y_rows_per_tile),
                       pl.ds(p * pw, pw)]
      y_dst = ysh.at[pl.ds(sid * y_rows_per_tile, y_rows_per_tile)]
      pltpu.async_copy(y_src, y_dst, ysem)
      if p > 0:
        pltpu.make_async_copy(
            zsh_rows, out_rows.at[:, pl.ds((p - 1) * pw, pw)], wsem).wait()
      for k in range(rows_per_tile // CHUNK):
        pltpu.sync_copy(zbuf, zsh.at[pl.ds(zrow0 + k * CHUNK, CHUNK)])
      pltpu.make_async_copy(y_src, y_dst, ysem).wait()
      plsc.subcore_barrier()

      gather(base, 0)

      @pl.when(cnt > 1)
      def _():
        gather(base + 1, 1)

      def body(jj, _):
        j = base + jj

        def step(b):
          bn = (b + 2) % 4
          gather_wait(j, b)
          scatter(j, b)

          @pl.when(jj + 2 < cnt)
          def _():
            @pl.when(jj >= 2)
            def _():
              scatter_wait(bn)

            gather(j + 2, bn)

        for b in range(4):
          @pl.when(jj % 4 == b)
          def _(b=b):
            step(b)

        return 0

      lax.fori_loop(0, cnt, body, 0)
      for b in range(4):
        @pl.when(cnt > b)
        def _(b=b):
          scatter_wait(b)

      plsc.subcore_barrier()
      pltpu.async_copy(zsh_rows, out_rows.at[:, pl.ds(p * pw, pw)], wsem)

    pltpu.make_async_copy(
        zsh_rows, out_rows.at[:, pl.ds((PASSES - 1) * pw, pw)], wsem).wait()

  return agg_kernel


def _stage_b_body(hist_ref, x_ref, w_ref, y_ref, dinv_ref):
  deg = hist_ref[0] + hist_ref[1] + 1.0
  dinv = lax.rsqrt(deg)
  y = jnp.dot(x_ref[...], w_ref[...], preferred_element_type=jnp.float32)
  y_ref[...] = y * dinv[:, 0:1]
  dinv_ref[...] = dinv


def _stage_d_body(z_ref, y1_ref, dinv_ref, w_ref, b_ref, y2_ref):
  d = dinv_ref[...][:, 0:1]
  agg = z_ref[0] + z_ref[1] + y1_ref[...]
  h = jnp.maximum(agg * d + b_ref[...], 0.0)
  y2_ref[...] = jnp.dot(h, w_ref[...], preferred_element_type=jnp.float32) * d


def _stage_f_body(z_ref, y2_ref, dinv_ref, b_ref, o_ref):
  d = dinv_ref[...][:, 0:1]
  o_ref[...] = (z_ref[0] + z_ref[1] + y2_ref[...]) * d + b_ref[...]


def kernel(x, edge_index, W1, b1, W2, b2):
  n, d_in = x.shape
  d_hid = W1.shape[1]
  d_out = W2.shape[1]
  e = edge_index.shape[1]

  cht = -(-e // (NS * CHUNK))
  ch0 = round(cht / 2)
  e_pad = NS * cht * CHUNK
  z_rows = -(-(n + 1) // (NS * CHUNK)) * NS * CHUNK

  row = edge_index[0].astype(jnp.int32)
  col = edge_index[1].astype(jnp.int32)
  pad = e_pad - e
  row_p = jnp.concatenate([row, jnp.zeros((pad,), jnp.int32)]).reshape(NS, cht, CHUNK)
  col_p = jnp.concatenate([col, jnp.full((pad,), n, jnp.int32)]).reshape(NS, cht, CHUNK)

  hist = _make_deg_kernel(cht, ch0, z_rows)(col_p)

  nblk = -(-n // BLK)
  full_spec = pl.BlockSpec((BLK, d_hid), lambda i: (i, 0))
  z_spec = pl.BlockSpec((NC, BLK, d_hid), lambda i: (0, i, 0))
  dinv_spec = pl.BlockSpec((BLK, 16), lambda i: (i, 0))

  y1, dinv = pl.pallas_call(
      _stage_b_body,
      grid=(nblk,),
      in_specs=[
          pl.BlockSpec((NC, BLK, 16), lambda i: (0, i, 0)),
          pl.BlockSpec((BLK, d_in), lambda i: (i, 0)),
          pl.BlockSpec((d_in, d_hid), lambda i: (0, 0)),
      ],
      out_specs=[full_spec, dinv_spec],
      out_shape=[
          jax.ShapeDtypeStruct((n, d_hid), jnp.float32),
          jax.ShapeDtypeStruct((n, 16), jnp.float32),
      ],
  )(hist, x, W1)

  agg_fn = _make_agg_kernel(n, d_hid, cht, ch0, z_rows)
  z1 = agg_fn(y1, row_p, col_p)

  y2 = pl.pallas_call(
      _stage_d_body,
      grid=(nblk,),
      in_specs=[
          z_spec, full_spec, dinv_spec,
          pl.BlockSpec((d_hid, d_out), lambda i: (0, 0)),
          pl.BlockSpec((1, d_out), lambda i: (0, 0)),
      ],
      out_specs=full_spec,
      out_shape=jax.ShapeDtypeStruct((n, d_out), jnp.float32),
  )(z1, y1, dinv, W2, b1.reshape(1, -1))

  z2 = agg_fn(y2, row_p, col_p)

  out = pl.pallas_call(
      _stage_f_body,
      grid=(nblk,),
      in_specs=[
          z_spec, full_spec, dinv_spec,
          pl.BlockSpec((1, d_out), lambda i: (0, 0)),
      ],
      out_specs=full_spec,
      out_shape=jax.ShapeDtypeStruct((n, d_out), jnp.float32),
  )(z2, y2, dinv, b2.reshape(1, -1))

  return out


# speedup vs baseline: 1.2485x; 1.0143x over previous
import functools

import jax
import jax.numpy as jnp
from jax import lax
from jax.experimental import pallas as pl
from jax.experimental.pallas import tpu as pltpu
from jax.experimental.pallas import tpu_sc as plsc

NC = 2
NS = 16
NW = NC * NS
CHUNK = 128
PASSES = 4
BLK = 2000


def _zero_fill(ref, nrows, width):
  zeros16 = jnp.zeros((16,), jnp.float32)

  def body(i, _):
    for j in range(width // 16):
      ref[i, pl.ds(j * 16, 16)] = zeros16
    return 0

  lax.fori_loop(0, nrows, body, 0)


def _fill_ones(ref, nrows, width):
  ones16 = jnp.ones((16,), jnp.float32)

  def body(i, _):
    for j in range(width // 16):
      ref[i, pl.ds(j * 16, 16)] = ones16
    return 0

  lax.fori_loop(0, nrows, body, 0)


@functools.lru_cache(maxsize=None)
def _make_deg_kernel(cht, ch0, z_rows):
  rows_per_tile = z_rows // NS
  mesh = plsc.VectorSubcoreMesh(core_axis_name="c", subcore_axis_name="s")

  @functools.partial(
      pl.kernel,
      out_type=jax.ShapeDtypeStruct((NC, z_rows, 16), jnp.float32),
      mesh=mesh,
      scratch_types=[
          pltpu.VMEM((cht, CHUNK), jnp.int32),
          pltpu.VMEM((CHUNK, 16), jnp.float32),
          pltpu.VMEM_SHARED((z_rows, 16), jnp.float32),
          pltpu.SemaphoreType.DMA,
          pltpu.SemaphoreType.DMA,
      ],
      compiler_params=pltpu.CompilerParams(use_tc_tiling_on_sc=False),
  )
  def deg_kernel(coli_hbm, out_hbm, coli_v, ones_v, hsh, isem, sem):
    cid = lax.axis_index("c")
    sid = lax.axis_index("s")
    base = jnp.where(cid == 0, 0, ch0)
    cnt = jnp.where(cid == 0, ch0, cht - ch0)
    pltpu.async_copy(coli_hbm.at[sid], coli_v, isem)
    _zero_fill(ones_v, CHUNK, 16)
    for k in range(rows_per_tile // CHUNK):
      pltpu.sync_copy(ones_v, hsh.at[pl.ds(sid * rows_per_tile + k * CHUNK, CHUNK)])
    _fill_ones(ones_v, CHUNK, 16)
    pltpu.make_async_copy(coli_hbm.at[sid], coli_v, isem).wait()
    plsc.subcore_barrier()

    def body(jj, _):
      pltpu.async_copy(ones_v, hsh.at[coli_v.at[base + jj]], sem, add=True)
      return 0

    lax.fori_loop(0, cnt, body, 0)

    def drain(jj, _):
      pltpu.make_async_copy(ones_v, hsh.at[coli_v.at[base]], sem).wait()
      return 0

    lax.fori_loop(0, cnt, drain, 0)
    plsc.subcore_barrier()
    pltpu.sync_copy(
        hsh.at[pl.ds(sid * rows_per_tile, rows_per_tile)],
        out_hbm.at[cid, pl.ds(sid * rows_per_tile, rows_per_tile)],
    )

  return deg_kernel


@functools.lru_cache(maxsize=None)
def _make_agg_kernel(n_rows, d, cht, ch0, z_rows):
  pw = d // PASSES
  rows_per_tile = z_rows // NS
  y_rows_per_tile = n_rows // NS
  mesh = plsc.VectorSubcoreMesh(core_axis_name="c", subcore_axis_name="s")

  @functools.partial(
      pl.kernel,
      out_type=jax.ShapeDtypeStruct((NC, z_rows, d), jnp.float32),
      mesh=mesh,
      scratch_types=[
          pltpu.VMEM((cht, CHUNK), jnp.int32),
          pltpu.VMEM((cht, CHUNK), jnp.int32),
          pltpu.VMEM((CHUNK, pw), jnp.float32),
          pltpu.VMEM((CHUNK, pw), jnp.float32),
          pltpu.VMEM((CHUNK, pw), jnp.float32),
          pltpu.VMEM((CHUNK, pw), jnp.float32),
          pltpu.VMEM((CHUNK, pw), jnp.float32),
          pltpu.VMEM_SHARED((z_rows, pw), jnp.float32),
          pltpu.VMEM_SHARED((n_rows, pw), jnp.float32),
          pltpu.SemaphoreType.DMA,
          pltpu.SemaphoreType.DMA,
          [pltpu.SemaphoreType.DMA] * 4,
          [pltpu.SemaphoreType.DMA] * 4,
      ],
      compiler_params=pltpu.CompilerParams(use_tc_tiling_on_sc=False),
  )
  def agg_kernel(y_hbm, rowi_hbm, coli_hbm, out_hbm,
                 rowi_v, coli_v, zbuf, gb0, gb1, gb2, gb3, zsh, ysh,
                 ysem, wsem, gsems, ssems):
    gbufs = (gb0, gb1, gb2, gb3)
    cid = lax.axis_index("c")
    sid = lax.axis_index("s")
    pltpu.async_copy(rowi_hbm.at[sid], rowi_v, gsems[0])
    pltpu.async_copy(coli_hbm.at[sid], coli_v, gsems[1])
    _zero_fill(zbuf, CHUNK, pw)
    pltpu.make_async_copy(rowi_hbm.at[sid], rowi_v, gsems[0]).wait()
    pltpu.make_async_copy(coli_hbm.at[sid], coli_v, gsems[1]).wait()
    base = jnp.where(cid == 0, 0, ch0)
    cnt = jnp.where(cid == 0, ch0, cht - ch0)

    def gather(j, b):
      pltpu.async_copy(ysh.at[rowi_v.at[j]], gbufs[b], gsems[b])

    def gather_wait(j, b):
      pltpu.make_async_copy(ysh.at[rowi_v.at[j]], gbufs[b], gsems[b]).wait()

    def scatter(j, b):
      pltpu.async_copy(gbufs[b], zsh.at[coli_v.at[j]], ssems[b], add=True)

    def scatter_wait(b):
      pltpu.make_async_copy(gbufs[b], zsh.at[coli_v.at[base]], ssems[b]).wait()

    zrow0 = sid * rows_per_tile
    out_rows = out_hbm.at[cid, pl.ds(zrow0, rows_per_tile)]
    zsh_rows = zsh.at[pl.ds(zrow0, rows_per_tile)]

    for p in range(PASSES):
      y_src = y_hbm.at[pl.ds(sid * y_rows_per_tile, y_rows_per_tile),
                       pl.ds(p * pw, pw)]
      y_dst = ysh.at[pl.ds(sid * y_rows_per_tile, y_rows_per_tile)]
      pltpu.async_copy(y_src, y_dst, ysem)
      if p > 0:
        pltpu.make_async_copy(
            zsh_rows, out_rows.at[:, pl.ds((p - 1) * pw, pw)], wsem).wait()
      for k in range(rows_per_tile // CHUNK):
        pltpu.sync_copy(zbuf, zsh.at[pl.ds(zrow0 + k * CHUNK, CHUNK)])
      pltpu.make_async_copy(y_src, y_dst, ysem).wait()
      plsc.subcore_barrier()

      gather(base, 0)

      @pl.when(cnt > 1)
      def _():
        gather(base + 1, 1)

      def body(jj, _):
        j = base + jj

        def step(b):
          bn = (b + 2) % 4
          gather_wait(j, b)
          scatter(j, b)

          @pl.when(jj + 2 < cnt)
          def _():
            @pl.when(jj >= 2)
            def _():
              scatter_wait(bn)

            gather(j + 2, bn)

        for b in range(4):
          @pl.when(jj % 4 == b)
          def _(b=b):
            step(b)

        return 0

      lax.fori_loop(0, cnt, body, 0)
      for b in range(4):
        @pl.when(cnt > b)
        def _(b=b):
          scatter_wait(b)

      plsc.subcore_barrier()
      pltpu.async_copy(zsh_rows, out_rows.at[:, pl.ds(p * pw, pw)], wsem)

    pltpu.make_async_copy(
        zsh_rows, out_rows.at[:, pl.ds((PASSES - 1) * pw, pw)], wsem).wait()

  return agg_kernel


def _stage_b_body(hist_ref, x_ref, w_ref, y_ref, dinv_ref):
  deg = hist_ref[0] + hist_ref[1] + 1.0
  dinv = lax.rsqrt(deg)
  y = jnp.dot(x_ref[...], w_ref[...], preferred_element_type=jnp.float32)
  y_ref[...] = y * dinv[:, 0:1]
  dinv_ref[...] = dinv


def _stage_d_body(z_ref, y1_ref, dinv_ref, w_ref, b_ref, y2_ref):
  d = dinv_ref[...][:, 0:1]
  agg = z_ref[0] + z_ref[1] + y1_ref[...]
  h = jnp.maximum(agg * d + b_ref[...], 0.0)
  y2_ref[...] = jnp.dot(h, w_ref[...], preferred_element_type=jnp.float32) * d


def _stage_f_body(z_ref, y2_ref, dinv_ref, b_ref, o_ref):
  d = dinv_ref[...][:, 0:1]
  o_ref[...] = (z_ref[0] + z_ref[1] + y2_ref[...]) * d + b_ref[...]


def kernel(x, edge_index, W1, b1, W2, b2):
  n, d_in = x.shape
  d_hid = W1.shape[1]
  d_out = W2.shape[1]
  e = edge_index.shape[1]

  cht = -(-e // (NS * CHUNK))
  ch0 = round(cht / 2)
  e_pad = NS * cht * CHUNK
  z_rows = -(-(n + 1) // (NS * CHUNK)) * NS * CHUNK

  row = edge_index[0].astype(jnp.int32)
  col = edge_index[1].astype(jnp.int32)
  pad = e_pad - e
  row_p = jnp.concatenate([row, jnp.zeros((pad,), jnp.int32)]).reshape(NS, cht, CHUNK)
  col_p = jnp.concatenate([col, jnp.full((pad,), n, jnp.int32)]).reshape(NS, cht, CHUNK)

  hist = _make_deg_kernel(cht, ch0, z_rows)(col_p)

  nblk = -(-n // BLK)
  full_spec = pl.BlockSpec((BLK, d_hid), lambda i: (i, 0))
  z_spec = pl.BlockSpec((NC, BLK, d_hid), lambda i: (0, i, 0))
  dinv_spec = pl.BlockSpec((BLK, 16), lambda i: (i, 0))

  y1, dinv = pl.pallas_call(
      _stage_b_body,
      grid=(nblk,),
      in_specs=[
          pl.BlockSpec((NC, BLK, 16), lambda i: (0, i, 0)),
          pl.BlockSpec((BLK, d_in), lambda i: (i, 0)),
          pl.BlockSpec((d_in, d_hid), lambda i: (0, 0)),
      ],
      out_specs=[full_spec, dinv_spec],
      out_shape=[
          jax.ShapeDtypeStruct((n, d_hid), jnp.float32),
          jax.ShapeDtypeStruct((n, 16), jnp.float32),
      ],
  )(hist, x, W1)

  agg_fn = _make_agg_kernel(n, d_hid, cht, ch0, z_rows)
  z1 = agg_fn(y1, row_p, col_p)

  y2 = pl.pallas_call(
      _stage_d_body,
      grid=(nblk,),
      in_specs=[
          z_spec, full_spec, dinv_spec,
          pl.BlockSpec((d_hid, d_out), lambda i: (0, 0)),
          pl.BlockSpec((1, d_out), lambda i: (0, 0)),
      ],
      out_specs=full_spec,
      out_shape=jax.ShapeDtypeStruct((n, d_out), jnp.float32),
  )(z1, y1, dinv, W2, b1.reshape(1, -1))

  z2 = agg_fn(y2, row_p, col_p)

  out = pl.pallas_call(
      _stage_f_body,
      grid=(nblk,),
      in_specs=[
          z_spec, full_spec, dinv_spec,
          pl.BlockSpec((1, d_out), lambda i: (0, 0)),
      ],
      out_specs=full_spec,
      out_shape=jax.ShapeDtypeStruct((n, d_out), jnp.float32),
  )(z2, y2, dinv, b2.reshape(1, -1))

  return out
